# baseline (device time: 44796 ns/iter reference)
import jax
import jax.numpy as jnp
from jax import lax
from jax.experimental import pallas as pl
from jax.experimental.pallas import tpu as pltpu

N_DEV = 8
N_CHUNKS = 4


def kernel(x, w_mat):
    m_loc, k = x.shape
    _, n = w_mat.shape
    n_loc = n // N_DEV
    c_loc = n // N_CHUNKS
    m_out = m_loc * N_DEV

    def body(x_ref, w_ref, out_ref, y_ref, qsend_ref, qrecv_ref,
             amax_src_ref, amax_ref, send_sems, recv_sems,
             a_send_sems, a_recv_sems):
        me = lax.axis_index("i")

        barrier = pltpu.get_barrier_semaphore()
        for off in range(1, N_DEV):
            dst = (me + off) % N_DEV
            pl.semaphore_signal(
                barrier, inc=1,
                device_id=(dst,), device_id_type=pl.DeviceIdType.MESH,
            )
        pl.semaphore_wait(barrier, N_DEV - 1)

        xb = x_ref[...].astype(jnp.bfloat16)
        amax = jnp.float32(0.0)
        for c in range(N_CHUNKS):
            wb = w_ref[:, c * c_loc:(c + 1) * c_loc].astype(jnp.bfloat16)
            yc = jnp.dot(xb, wb, preferred_element_type=jnp.float32)
            amax = jnp.maximum(amax, jnp.max(jnp.abs(yc)))
            y_ref[:, c * c_loc:(c + 1) * c_loc] = yc

        amax_src_ref[...] = jnp.full((1, 128), amax, jnp.float32)
        a_rdmas = []
        for off in range(1, N_DEV):
            dst = (me + off) % N_DEV
            ard = pltpu.make_async_remote_copy(
                src_ref=amax_src_ref,
                dst_ref=amax_ref.at[pl.ds(off - 1, 1)],
                send_sem=a_send_sems.at[off - 1],
                recv_sem=a_recv_sems.at[off - 1],
                device_id=(dst,),
                device_id_type=pl.DeviceIdType.MESH,
            )
            ard.start()
            a_rdmas.append(ard)
        for ard in a_rdmas:
            ard.wait_recv()
        gmax = jnp.maximum(amax, jnp.max(amax_ref[...]))
        scale = gmax / 448.0

        rdmas = []
        for off in range(1, N_DEV):
            dst = (me + off) % N_DEV
            yblk = y_ref[:, pl.ds(dst * n_loc, n_loc)]
            qsend_ref[off - 1] = (yblk / scale).astype(jnp.float8_e4m3fn)
            rd = pltpu.make_async_remote_copy(
                src_ref=qsend_ref.at[off - 1],
                dst_ref=qrecv_ref.at[off - 1],
                send_sem=send_sems.at[off - 1],
                recv_sem=recv_sems.at[off - 1],
                device_id=(dst,),
                device_id_type=pl.DeviceIdType.MESH,
            )
            rd.start()
            rdmas.append(rd)

        q_own = (y_ref[:, pl.ds(me * n_loc, n_loc)] / scale).astype(
            jnp.float8_e4m3fn)
        out_ref[pl.ds(me * m_loc, m_loc), :] = (
            q_own.astype(jnp.float32) * scale)

        for off in range(1, N_DEV):
            src = (me - off) % N_DEV
            rdmas[off - 1].wait_recv()
            out_ref[pl.ds(src * m_loc, m_loc), :] = (
                qrecv_ref[off - 1].astype(jnp.float32) * scale)

        for rd in rdmas:
            rd.wait_send()
        for ard in a_rdmas:
            ard.wait_send()

    return pl.pallas_call(
        body,
        out_shape=jax.ShapeDtypeStruct((m_out, n_loc), jnp.float32),
        in_specs=[
            pl.BlockSpec(memory_space=pltpu.VMEM),
            pl.BlockSpec(memory_space=pltpu.VMEM),
        ],
        out_specs=pl.BlockSpec(memory_space=pltpu.VMEM),
        scratch_shapes=[
            pltpu.VMEM((m_loc, n), jnp.float32),
            pltpu.VMEM((N_DEV - 1, m_loc, n_loc), jnp.float8_e4m3fn),
            pltpu.VMEM((N_DEV - 1, m_loc, n_loc), jnp.float8_e4m3fn),
            pltpu.VMEM((1, 128), jnp.float32),
            pltpu.VMEM((N_DEV - 1, 128), jnp.float32),
            pltpu.SemaphoreType.DMA((N_DEV - 1,)),
            pltpu.SemaphoreType.DMA((N_DEV - 1,)),
            pltpu.SemaphoreType.DMA((N_DEV - 1,)),
            pltpu.SemaphoreType.DMA((N_DEV - 1,)),
        ],
        compiler_params=pltpu.CompilerParams(
            collective_id=0,
            vmem_limit_bytes=60 * 1024 * 1024,
        ),
    )(x, w_mat)


# device time: 40047 ns/iter; 1.1186x vs baseline; 1.1186x over previous
import jax
import jax.numpy as jnp
from jax import lax
from jax.experimental import pallas as pl
from jax.experimental.pallas import tpu as pltpu

N_DEV = 8
N_CHUNKS = 4


def kernel(x, w_mat):
    m_loc, k = x.shape
    _, n = w_mat.shape
    n_loc = n // N_DEV
    c_loc = n // N_CHUNKS
    m_out = m_loc * N_DEV

    def body(x_ref, w_ref, out_ref, xb_ref, y_ref, amax_acc_ref,
             qsend_ref, qrecv_ref, amax_src_ref, amax_ref,
             send_sems, recv_sems, a_send_sems, a_recv_sems):
        c = pl.program_id(0)
        me = lax.axis_index("i")

        @pl.when(c == 0)
        def _prologue():
            barrier = pltpu.get_barrier_semaphore()
            for off in range(1, N_DEV):
                dst = (me + off) % N_DEV
                pl.semaphore_signal(
                    barrier, inc=1,
                    device_id=(dst,), device_id_type=pl.DeviceIdType.MESH,
                )
            pl.semaphore_wait(barrier, N_DEV - 1)
            xb_ref[...] = x_ref[...].astype(jnp.bfloat16)
            amax_acc_ref[...] = jnp.zeros((1, 128), jnp.float32)

        wb = w_ref[...].astype(jnp.bfloat16)
        yc = jnp.dot(xb_ref[...], wb, preferred_element_type=jnp.float32)
        amax_acc_ref[...] = jnp.maximum(
            amax_acc_ref[...], jnp.max(jnp.abs(yc)))
        y_ref[:, pl.ds(c * c_loc, c_loc)] = yc

        @pl.when(c == N_CHUNKS - 1)
        def _epilogue():
            amax = jnp.max(amax_acc_ref[...])

            amax_src_ref[...] = jnp.full((1, 128), amax, jnp.float32)
            a_rdmas = []
            for off in range(1, N_DEV):
                dst = (me + off) % N_DEV
                ard = pltpu.make_async_remote_copy(
                    src_ref=amax_src_ref,
                    dst_ref=amax_ref.at[pl.ds(off - 1, 1)],
                    send_sem=a_send_sems.at[off - 1],
                    recv_sem=a_recv_sems.at[off - 1],
                    device_id=(dst,),
                    device_id_type=pl.DeviceIdType.MESH,
                )
                ard.start()
                a_rdmas.append(ard)
            for ard in a_rdmas:
                ard.wait_recv()
            gmax = jnp.maximum(amax, jnp.max(amax_ref[...]))
            scale = gmax / 448.0

            rdmas = []
            for off in range(1, N_DEV):
                dst = (me + off) % N_DEV
                yblk = y_ref[:, pl.ds(dst * n_loc, n_loc)]
                qsend_ref[off - 1] = (yblk / scale).astype(jnp.float8_e4m3fn)
                rd = pltpu.make_async_remote_copy(
                    src_ref=qsend_ref.at[off - 1],
                    dst_ref=qrecv_ref.at[off - 1],
                    send_sem=send_sems.at[off - 1],
                    recv_sem=recv_sems.at[off - 1],
                    device_id=(dst,),
                    device_id_type=pl.DeviceIdType.MESH,
                )
                rd.start()
                rdmas.append(rd)

            q_own = (y_ref[:, pl.ds(me * n_loc, n_loc)] / scale).astype(
                jnp.float8_e4m3fn)
            out_ref[pl.ds(me * m_loc, m_loc), :] = (
                q_own.astype(jnp.float32) * scale)

            for off in range(1, N_DEV):
                src = (me - off) % N_DEV
                rdmas[off - 1].wait_recv()
                out_ref[pl.ds(src * m_loc, m_loc), :] = (
                    qrecv_ref[off - 1].astype(jnp.float32) * scale)

            for rd in rdmas:
                rd.wait_send()
            for ard in a_rdmas:
                ard.wait_send()

    grid_spec = pltpu.PrefetchScalarGridSpec(
        num_scalar_prefetch=0,
        grid=(N_CHUNKS,),
        in_specs=[
            pl.BlockSpec((m_loc, k), lambda c: (0, 0)),
            pl.BlockSpec((k, c_loc), lambda c: (0, c)),
        ],
        out_specs=pl.BlockSpec((m_out, n_loc), lambda c: (0, 0)),
        scratch_shapes=[
            pltpu.VMEM((m_loc, k), jnp.bfloat16),
            pltpu.VMEM((m_loc, n), jnp.float32),
            pltpu.VMEM((1, 128), jnp.float32),
            pltpu.VMEM((N_DEV - 1, m_loc, n_loc), jnp.float8_e4m3fn),
            pltpu.VMEM((N_DEV - 1, m_loc, n_loc), jnp.float8_e4m3fn),
            pltpu.VMEM((1, 128), jnp.float32),
            pltpu.VMEM((N_DEV - 1, 128), jnp.float32),
            pltpu.SemaphoreType.DMA((N_DEV - 1,)),
            pltpu.SemaphoreType.DMA((N_DEV - 1,)),
            pltpu.SemaphoreType.DMA((N_DEV - 1,)),
            pltpu.SemaphoreType.DMA((N_DEV - 1,)),
        ],
    )

    return pl.pallas_call(
        body,
        out_shape=jax.ShapeDtypeStruct((m_out, n_loc), jnp.float32),
        grid_spec=grid_spec,
        compiler_params=pltpu.CompilerParams(
            collective_id=0,
            vmem_limit_bytes=60 * 1024 * 1024,
        ),
    )(x, w_mat)
